# Initial kernel scaffold; baseline (speedup 1.0000x reference)
import functools

import jax
import jax.numpy as jnp
from jax import lax
from jax.experimental import pallas as pl
from jax.experimental.pallas import tpu as pltpu

B, N, D = 2, 4096, 1024
F = 8192
THRESH = 0.3
BM_PTR = 1024
BM_C = 512


def _ptr_body(h_ref, W1_ref, b1_ref, w2_ref, b2_ref, p0_ref, p1_ref, p2_ref):
    blocks_per_batch = N // BM_PTR
    row_off = (pl.program_id(0) // blocks_per_batch) * N
    h = h_ref[...]
    W1 = W1_ref[...]
    b1 = b1_ref[...]
    w2 = w2_ref[...]
    b2 = b2_ref[0, 0]
    outs = (p0_ref, p1_ref, p2_ref)
    nb = None
    for b in range(3):
        u = jnp.dot(h + jnp.float32(0.1 * b), W1,
                    preferred_element_type=jnp.float32) + b1
        gl = jax.nn.gelu(u, approximate=False)
        logit = jnp.sum(gl * w2, axis=1, keepdims=True) + b2
        s = jax.nn.sigmoid(logit)
        tgt = jnp.clip(jnp.round(s * (N - 1)).astype(jnp.int32), 0, N - 1)
        if b == 0:
            nb = jnp.clip(jnp.round(s / THRESH).astype(jnp.int32), 1, 3)
            ptr = tgt
        else:
            ptr = jnp.where(b < nb, tgt, 0)
        outs[b][...] = ptr + row_off


def _compute_pointers(hf, W1, b1, W2, b2):
    grid = (F // BM_PTR,)
    out = pl.pallas_call(
        _ptr_body,
        grid=grid,
        in_specs=[
            pl.BlockSpec((BM_PTR, D), lambda i: (i, 0)),
            pl.BlockSpec((D, D // 2), lambda i: (0, 0)),
            pl.BlockSpec((1, D // 2), lambda i: (0, 0)),
            pl.BlockSpec((1, D // 2), lambda i: (0, 0)),
            pl.BlockSpec((1, 1), lambda i: (0, 0)),
        ],
        out_specs=[pl.BlockSpec((BM_PTR, 1), lambda i: (i, 0))] * 3,
        out_shape=[jax.ShapeDtypeStruct((F, 1), jnp.int32)] * 3,
    )(hf, W1, b1.reshape(1, -1), W2.reshape(1, -1), b2.reshape(1, 1))
    return out


def _prep_body(Wv_ref, Wt1_ref, Wt2_ref, Wo_ref, bt2_ref,
               M1_ref, M2_ref, M3_ref, c0_ref):
    Wv = Wv_ref[...]
    Wo = Wo_ref[...]
    M1_ref[...] = jnp.dot(Wv, Wt1_ref[:D, :], preferred_element_type=jnp.float32)
    M2_ref[...] = jnp.dot(Wv, Wt1_ref[D:, :], preferred_element_type=jnp.float32)
    M3_ref[...] = jnp.dot(Wt2_ref[...], Wo, preferred_element_type=jnp.float32)
    c0_ref[...] = jnp.dot(bt2_ref[...], Wo, preferred_element_type=jnp.float32)


def _prep_weights(Wv, Wt1, Wt2, Wo, bt2):
    return pl.pallas_call(
        _prep_body,
        out_shape=[
            jax.ShapeDtypeStruct((D, D), jnp.float32),
            jax.ShapeDtypeStruct((D, D), jnp.float32),
            jax.ShapeDtypeStruct((D, D), jnp.float32),
            jax.ShapeDtypeStruct((1, D), jnp.float32),
        ],
    )(Wv, Wt1, Wt2, Wo, bt2.reshape(1, -1))


def _dense_body(h_ref, g_ref, M1_ref, M2_ref, M3_ref, bt1_ref, c0_ref, o_ref):
    pre = (jnp.dot(h_ref[...], M1_ref[...], preferred_element_type=jnp.float32)
           + jnp.dot(g_ref[...], M2_ref[...], preferred_element_type=jnp.float32)
           + bt1_ref[...])
    act = jax.nn.gelu(pre, approximate=False)
    o_ref[...] = jnp.dot(act, M3_ref[...],
                         preferred_element_type=jnp.float32) + c0_ref[...]


def _dense(hf, g, M1, M2, M3, bt1, c0):
    grid = (F // BM_C,)
    return pl.pallas_call(
        _dense_body,
        grid=grid,
        in_specs=[
            pl.BlockSpec((BM_C, D), lambda i: (i, 0)),
            pl.BlockSpec((BM_C, D), lambda i: (i, 0)),
            pl.BlockSpec((D, D), lambda i: (0, 0)),
            pl.BlockSpec((D, D), lambda i: (0, 0)),
            pl.BlockSpec((D, D), lambda i: (0, 0)),
            pl.BlockSpec((1, D), lambda i: (0, 0)),
            pl.BlockSpec((1, D), lambda i: (0, 0)),
        ],
        out_specs=pl.BlockSpec((BM_C, D), lambda i: (i, 0)),
        out_shape=jax.ShapeDtypeStruct((F, D), jnp.float32),
    )(hf, g, M1, M2, M3, bt1.reshape(1, -1), c0)


def kernel(h, W1, b1, W2, b2, Wv, Wt1, bt1, Wt2, bt2, Wo):
    hf = h.reshape(F, D)
    p0, p1, p2 = _compute_pointers(hf, W1, b1, W2, b2)
    i0, i1, i2 = p0.reshape(F), p1.reshape(F), p2.reshape(F)
    g = (jnp.take(hf, i0, axis=0) + jnp.take(hf, i1, axis=0)
         + jnp.take(hf, i2, axis=0)) * jnp.float32(1.0 / 3.0)
    M1, M2, M3, c0 = _prep_weights(Wv, Wt1, Wt2, Wo, bt2)
    z = _dense(hf, g, M1, M2, M3, bt1, c0)
    return z.reshape(B, N, D)


# speedup vs baseline: 2.1527x; 2.1527x over previous
import functools

import jax
import jax.numpy as jnp
from jax import lax
from jax.experimental import pallas as pl
from jax.experimental.pallas import tpu as pltpu

def _gelu_exact(x):
    return 0.5 * x * (1.0 + lax.erf(x * jnp.float32(0.7071067811865476)))


B, N, D = 2, 4096, 1024
F = 8192
THRESH = 0.3
BM_PTR = 1024
BM_C = 512


def _ptr_body(h_ref, W1_ref, b1_ref, w2_ref, b2_ref, p0_ref, p1_ref, p2_ref, hb_ref):
    blocks_per_batch = N // BM_PTR
    row_off = (pl.program_id(0) // blocks_per_batch) * N
    h = h_ref[...]
    W1 = W1_ref[...].astype(jnp.bfloat16)
    b1 = b1_ref[...]
    w2 = w2_ref[...].astype(jnp.bfloat16)
    b2 = b2_ref[0, 0]
    outs = (p0_ref, p1_ref, p2_ref)
    hb_ref[...] = h.astype(jnp.bfloat16)
    nb = None
    for b in range(3):
        hb = h if b == 0 else h + jnp.float32(0.1 * b)
        u = jnp.dot(hb.astype(jnp.bfloat16), W1,
                    preferred_element_type=jnp.float32) + b1
        gl = _gelu_exact(u)
        logit = jnp.dot(gl.astype(jnp.bfloat16), w2,
                        preferred_element_type=jnp.float32) + b2
        s = jax.nn.sigmoid(logit)
        tgt = jnp.clip(jnp.round(s * (N - 1)).astype(jnp.int32), 0, N - 1)
        if b == 0:
            nb = jnp.clip(jnp.round(s / THRESH).astype(jnp.int32), 1, 3)
            ptr = tgt
        else:
            ptr = jnp.where(b < nb, tgt, 0)
        outs[b][...] = ptr + row_off


def _compute_pointers(hf, W1, b1, W2, b2):
    grid = (F // BM_PTR,)
    out = pl.pallas_call(
        _ptr_body,
        grid=grid,
        in_specs=[
            pl.BlockSpec((BM_PTR, D), lambda i: (i, 0)),
            pl.BlockSpec((D, D // 2), lambda i: (0, 0)),
            pl.BlockSpec((1, D // 2), lambda i: (0, 0)),
            pl.BlockSpec((D // 2, 1), lambda i: (0, 0)),
            pl.BlockSpec((1, 1), lambda i: (0, 0)),
        ],
        out_specs=[pl.BlockSpec((BM_PTR, 1), lambda i: (i, 0))] * 3
        + [pl.BlockSpec((BM_PTR, D), lambda i: (i, 0))],
        out_shape=[jax.ShapeDtypeStruct((F, 1), jnp.int32)] * 3
        + [jax.ShapeDtypeStruct((F, D), jnp.bfloat16)],
    )(hf, W1, b1.reshape(1, -1), W2, b2.reshape(1, 1))
    return out


def _prep_body(Wv_ref, Wt1_ref, Wt2_ref, Wo_ref, bt2_ref,
               M1_ref, M2_ref, M3_ref, c0_ref):
    Wv = Wv_ref[...].astype(jnp.bfloat16)
    Wo = Wo_ref[...].astype(jnp.bfloat16)
    M1_ref[...] = jnp.dot(Wv, Wt1_ref[:D, :].astype(jnp.bfloat16), preferred_element_type=jnp.float32)
    M2_ref[...] = jnp.dot(Wv, Wt1_ref[D:, :].astype(jnp.bfloat16), preferred_element_type=jnp.float32)
    M3_ref[...] = jnp.dot(Wt2_ref[...].astype(jnp.bfloat16), Wo, preferred_element_type=jnp.float32)
    c0_ref[...] = jnp.dot(bt2_ref[...].astype(jnp.bfloat16), Wo, preferred_element_type=jnp.float32)


def _prep_weights(Wv, Wt1, Wt2, Wo, bt2):
    return pl.pallas_call(
        _prep_body,
        out_shape=[
            jax.ShapeDtypeStruct((D, D), jnp.float32),
            jax.ShapeDtypeStruct((D, D), jnp.float32),
            jax.ShapeDtypeStruct((D, D), jnp.float32),
            jax.ShapeDtypeStruct((1, D), jnp.float32),
        ],
    )(Wv, Wt1, Wt2, Wo, bt2.reshape(1, -1))


def _dense_body(h_ref, g_ref, M1_ref, M2_ref, M3_ref, bt1_ref, c0_ref, o_ref):
    pre = (jnp.dot(h_ref[...], M1_ref[...].astype(jnp.bfloat16),
                   preferred_element_type=jnp.float32)
           + jnp.dot(g_ref[...], M2_ref[...].astype(jnp.bfloat16),
                     preferred_element_type=jnp.float32)
           + bt1_ref[...])
    act = _gelu_exact(pre)
    o_ref[...] = jnp.dot(act.astype(jnp.bfloat16), M3_ref[...].astype(jnp.bfloat16),
                         preferred_element_type=jnp.float32) + c0_ref[...]


def _dense(hf, g, M1, M2, M3, bt1, c0):
    grid = (F // BM_C,)
    return pl.pallas_call(
        _dense_body,
        grid=grid,
        in_specs=[
            pl.BlockSpec((BM_C, D), lambda i: (i, 0)),
            pl.BlockSpec((BM_C, D), lambda i: (i, 0)),
            pl.BlockSpec((D, D), lambda i: (0, 0)),
            pl.BlockSpec((D, D), lambda i: (0, 0)),
            pl.BlockSpec((D, D), lambda i: (0, 0)),
            pl.BlockSpec((1, D), lambda i: (0, 0)),
            pl.BlockSpec((1, D), lambda i: (0, 0)),
        ],
        out_specs=pl.BlockSpec((BM_C, D), lambda i: (i, 0)),
        out_shape=jax.ShapeDtypeStruct((F, D), jnp.float32),
    )(hf, g, M1, M2, M3, bt1.reshape(1, -1), c0)


try:
    from jax.experimental.pallas import tpu_sc as plsc
    _SC_INFO = None

    def _sc_info():
        global _SC_INFO
        if _SC_INFO is None:
            _SC_INFO = plsc.get_sparse_core_info()
        return _SC_INFO
except ImportError:
    plsc = None

_CHUNK = 16


def _gather_avg_body(tab_hbm, i0_hbm, i1_hbm, i2_hbm, out_hbm,
                     idx0_v, idx1_v, idx2_v, b0, b1, b2, sem):
    nc = 2
    wid = lax.axis_index("s") * nc + lax.axis_index("c")
    nw = 32
    rows_per_w = F // nw
    nchunks = rows_per_w // _CHUNK
    row0 = wid * rows_per_w

    def chunk_body(k, _):
        base = pl.multiple_of(row0 + k * _CHUNK, _CHUNK)
        pltpu.sync_copy(i0_hbm.at[pl.ds(base, _CHUNK)], idx0_v)
        pltpu.sync_copy(i1_hbm.at[pl.ds(base, _CHUNK)], idx1_v)
        pltpu.sync_copy(i2_hbm.at[pl.ds(base, _CHUNK)], idx2_v)
        c0 = pltpu.async_copy(tab_hbm.at[idx0_v], b0, sem)
        c1 = pltpu.async_copy(tab_hbm.at[idx1_v], b1, sem)
        c2 = pltpu.async_copy(tab_hbm.at[idx2_v], b2, sem)
        c0.wait()
        c1.wait()
        c2.wait()

        third = jnp.full((16,), 1.0 / 3.0, dtype=jnp.float32)
        himask = jnp.full((16,), -65536, dtype=jnp.int32)
        rbias = jnp.full((16,), 0x7FFF, dtype=jnp.int32)
        one = jnp.full((16,), 1, dtype=jnp.int32)

        def _lo_f32(w):
            return lax.bitcast_convert_type(lax.shift_left(w, 16), jnp.float32)

        def _hi_f32(w):
            return lax.bitcast_convert_type(w & himask, jnp.float32)

        def _rtne_bits(x):
            u = lax.bitcast_convert_type(x, jnp.int32)
            r = u + rbias + (lax.shift_right_logical(u, 16) & one)
            return r & himask

        for j in range(_CHUNK):
            def grp_body(i, _, j=j):
                off = pl.multiple_of(i * 16, 16)
                w0 = b0[j, pl.ds(off, 16)]
                w1 = b1[j, pl.ds(off, 16)]
                w2 = b2[j, pl.ds(off, 16)]
                s_lo = (_lo_f32(w0) + _lo_f32(w1) + _lo_f32(w2)) * third
                s_hi = (_hi_f32(w0) + _hi_f32(w1) + _hi_f32(w2)) * third
                packed = (lax.shift_right_logical(_rtne_bits(s_lo), 16)
                          | _rtne_bits(s_hi))
                b0[j, pl.ds(off, 16)] = packed
                return 0

            lax.fori_loop(0, (D // 2) // 16, grp_body, 0, unroll=4)
        pltpu.sync_copy(b0, out_hbm.at[pl.ds(base, _CHUNK)])
        return 0

    lax.fori_loop(0, nchunks, chunk_body, 0)


def _gather_avg(hf, i0, i1, i2):
    import functools as _ft
    mesh = plsc.VectorSubcoreMesh(core_axis_name="c", subcore_axis_name="s")
    kfn = _ft.partial(
        pl.kernel,
        mesh=mesh,
        out_type=jax.ShapeDtypeStruct((F, D // 2), jnp.int32),
        scratch_types=[
            pltpu.VMEM((_CHUNK,), jnp.int32),
            pltpu.VMEM((_CHUNK,), jnp.int32),
            pltpu.VMEM((_CHUNK,), jnp.int32),
            pltpu.VMEM((_CHUNK, D // 2), jnp.int32),
            pltpu.VMEM((_CHUNK, D // 2), jnp.int32),
            pltpu.VMEM((_CHUNK, D // 2), jnp.int32),
            pltpu.SemaphoreType.DMA,
        ],
    )(_gather_avg_body)
    return kfn(hf, i0, i1, i2)


def kernel(h, W1, b1, W2, b2, Wv, Wt1, bt1, Wt2, bt2, Wo):
    hf = h.reshape(F, D)
    p0, p1, p2, hb16 = _compute_pointers(hf, W1, b1, W2, b2)
    i0, i1, i2 = p0.reshape(F), p1.reshape(F), p2.reshape(F)
    tab = lax.bitcast_convert_type(hb16.reshape(F, D // 2, 2), jnp.int32)
    gi = _gather_avg(tab, i0, i1, i2)
    g = lax.bitcast_convert_type(gi, jnp.bfloat16).reshape(F, D)
    M1, M2, M3, c0 = _prep_weights(Wv, Wt1, Wt2, Wo, bt2)
    z = _dense(hb16, g, M1, M2, M3, bt1, c0)
    return z.reshape(B, N, D)


# speedup vs baseline: 3.9012x; 1.8122x over previous
import functools

import jax
import jax.numpy as jnp
from jax import lax
from jax.experimental import pallas as pl
from jax.experimental.pallas import tpu as pltpu

def _gelu_exact(x):
    return 0.5 * x * (1.0 + lax.erf(x * jnp.float32(0.7071067811865476)))


B, N, D = 2, 4096, 1024
F = 8192
THRESH = 0.3
BM_PTR = 1024
BM_C = 512


def _ptr_body(h_ref, W1_ref, b1_ref, w2_ref, b2_ref, p0_ref, p1_ref, p2_ref, tab_ref):
    blocks_per_batch = N // BM_PTR
    row_off = (pl.program_id(0) // blocks_per_batch) * N
    h = h_ref[...]
    W1 = W1_ref[...].astype(jnp.bfloat16)
    b1 = b1_ref[...]
    w2 = w2_ref[...].astype(jnp.bfloat16)
    b2 = b2_ref[0, 0]
    outs = (p0_ref, p1_ref, p2_ref)
    u = lax.bitcast_convert_type(h, jnp.int32)
    r = u + 0x7FFF + (lax.shift_right_logical(u, 16) & 1)
    tab_ref[...] = (lax.shift_right_logical(r[:, :D // 2], 16)
                    | (r[:, D // 2:] & jnp.int32(-65536)))
    nb = None
    for b in range(3):
        hb = h if b == 0 else h + jnp.float32(0.1 * b)
        u = jnp.dot(hb.astype(jnp.bfloat16), W1,
                    preferred_element_type=jnp.float32) + b1
        gl = _gelu_exact(u)
        logit = jnp.dot(gl.astype(jnp.bfloat16), w2,
                        preferred_element_type=jnp.float32) + b2
        s = jax.nn.sigmoid(logit)
        tgt = jnp.clip(jnp.round(s * (N - 1)).astype(jnp.int32), 0, N - 1)
        if b == 0:
            nb = jnp.clip(jnp.round(s / THRESH).astype(jnp.int32), 1, 3)
            ptr = tgt
        else:
            ptr = jnp.where(b < nb, tgt, 0)
        outs[b][...] = ptr + row_off


def _compute_pointers(hf, W1, b1, W2, b2):
    grid = (F // BM_PTR,)
    out = pl.pallas_call(
        _ptr_body,
        grid=grid,
        in_specs=[
            pl.BlockSpec((BM_PTR, D), lambda i: (i, 0)),
            pl.BlockSpec((D, D // 2), lambda i: (0, 0)),
            pl.BlockSpec((1, D // 2), lambda i: (0, 0)),
            pl.BlockSpec((D // 2, 1), lambda i: (0, 0)),
            pl.BlockSpec((1, 1), lambda i: (0, 0)),
        ],
        out_specs=[pl.BlockSpec((BM_PTR, 1), lambda i: (i, 0))] * 3
        + [pl.BlockSpec((BM_PTR, D // 2), lambda i: (i, 0))],
        out_shape=[jax.ShapeDtypeStruct((F, 1), jnp.int32)] * 3
        + [jax.ShapeDtypeStruct((F, D // 2), jnp.int32)],
    )(hf, W1, b1.reshape(1, -1), W2, b2.reshape(1, 1))
    return out


def _prep_body(Wv_ref, Wt1_ref, Wt2_ref, Wo_ref, bt2_ref,
               M1_ref, M2_ref, M3_ref, c0_ref):
    Wv = Wv_ref[...].astype(jnp.bfloat16)
    Wo = Wo_ref[...].astype(jnp.bfloat16)
    M1_ref[...] = jnp.dot(Wv, Wt1_ref[:D, :].astype(jnp.bfloat16), preferred_element_type=jnp.float32)
    M2_ref[...] = jnp.dot(Wv, Wt1_ref[D:, :].astype(jnp.bfloat16), preferred_element_type=jnp.float32)
    M3_ref[...] = jnp.dot(Wt2_ref[...].astype(jnp.bfloat16), Wo, preferred_element_type=jnp.float32)
    c0_ref[...] = jnp.dot(bt2_ref[...].astype(jnp.bfloat16), Wo, preferred_element_type=jnp.float32)


def _prep_weights(Wv, Wt1, Wt2, Wo, bt2):
    return pl.pallas_call(
        _prep_body,
        out_shape=[
            jax.ShapeDtypeStruct((D, D), jnp.float32),
            jax.ShapeDtypeStruct((D, D), jnp.float32),
            jax.ShapeDtypeStruct((D, D), jnp.float32),
            jax.ShapeDtypeStruct((1, D), jnp.float32),
        ],
    )(Wv, Wt1, Wt2, Wo, bt2.reshape(1, -1))


def _unlo(w):
    return lax.bitcast_convert_type(lax.shift_left(w, 16), jnp.float32)


def _unhi(w):
    return lax.bitcast_convert_type(w & jnp.int32(-65536), jnp.float32)


def _dense_body(h_ref, g_ref, M1_ref, M2_ref, M3_ref, bt1_ref, c0_ref, o_ref):
    t = h_ref[...]
    gw = g_ref[...]
    Dh = D // 2
    pre = (jnp.dot(_unlo(t).astype(jnp.bfloat16),
                   M1_ref[:Dh, :].astype(jnp.bfloat16),
                   preferred_element_type=jnp.float32)
           + jnp.dot(_unhi(t).astype(jnp.bfloat16),
                     M1_ref[Dh:, :].astype(jnp.bfloat16),
                     preferred_element_type=jnp.float32)
           + jnp.dot(_unlo(gw).astype(jnp.bfloat16),
                     M2_ref[:Dh, :].astype(jnp.bfloat16),
                     preferred_element_type=jnp.float32)
           + jnp.dot(_unhi(gw).astype(jnp.bfloat16),
                     M2_ref[Dh:, :].astype(jnp.bfloat16),
                     preferred_element_type=jnp.float32)
           + bt1_ref[...])
    act = _gelu_exact(pre)
    o_ref[...] = jnp.dot(act.astype(jnp.bfloat16), M3_ref[...].astype(jnp.bfloat16),
                         preferred_element_type=jnp.float32) + c0_ref[...]


def _dense(hf, g, M1, M2, M3, bt1, c0):
    grid = (F // BM_C,)
    return pl.pallas_call(
        _dense_body,
        grid=grid,
        in_specs=[
            pl.BlockSpec((BM_C, D // 2), lambda i: (i, 0)),
            pl.BlockSpec((BM_C, D // 2), lambda i: (i, 0)),
            pl.BlockSpec((D, D), lambda i: (0, 0)),
            pl.BlockSpec((D, D), lambda i: (0, 0)),
            pl.BlockSpec((D, D), lambda i: (0, 0)),
            pl.BlockSpec((1, D), lambda i: (0, 0)),
            pl.BlockSpec((1, D), lambda i: (0, 0)),
        ],
        out_specs=pl.BlockSpec((BM_C, D), lambda i: (i, 0)),
        out_shape=jax.ShapeDtypeStruct((F, D), jnp.float32),
    )(hf, g, M1, M2, M3, bt1.reshape(1, -1), c0)


try:
    from jax.experimental.pallas import tpu_sc as plsc
    _SC_INFO = None

    def _sc_info():
        global _SC_INFO
        if _SC_INFO is None:
            _SC_INFO = plsc.get_sparse_core_info()
        return _SC_INFO
except ImportError:
    plsc = None

_CHUNK = 16


def _gather_avg_body(tab_hbm, i0_hbm, i1_hbm, i2_hbm, out_hbm,
                     idx0_v, idx1_v, idx2_v, b0, b1, b2, sem):
    nc = 2
    wid = lax.axis_index("s") * nc + lax.axis_index("c")
    nw = 32
    rows_per_w = F // nw
    nchunks = rows_per_w // _CHUNK
    row0 = wid * rows_per_w

    pltpu.sync_copy(i0_hbm.at[pl.ds(row0, rows_per_w)], idx0_v)
    pltpu.sync_copy(i1_hbm.at[pl.ds(row0, rows_per_w)], idx1_v)
    pltpu.sync_copy(i2_hbm.at[pl.ds(row0, rows_per_w)], idx2_v)

    def chunk_body(k, _):
        base = pl.multiple_of(row0 + k * _CHUNK, _CHUNK)
        koff = pl.multiple_of(k * _CHUNK, _CHUNK)
        c0 = pltpu.async_copy(tab_hbm.at[idx0_v.at[pl.ds(koff, _CHUNK)]], b0, sem)
        c1 = pltpu.async_copy(tab_hbm.at[idx1_v.at[pl.ds(koff, _CHUNK)]], b1, sem)
        c2 = pltpu.async_copy(tab_hbm.at[idx2_v.at[pl.ds(koff, _CHUNK)]], b2, sem)
        c0.wait()
        c1.wait()
        c2.wait()

        third = jnp.full((16,), 1.0 / 3.0, dtype=jnp.float32)
        himask = jnp.full((16,), -65536, dtype=jnp.int32)
        rbias = jnp.full((16,), 0x7FFF, dtype=jnp.int32)
        one = jnp.full((16,), 1, dtype=jnp.int32)

        def _lo_f32(w):
            return lax.bitcast_convert_type(lax.shift_left(w, 16), jnp.float32)

        def _hi_f32(w):
            return lax.bitcast_convert_type(w & himask, jnp.float32)

        def _rtne_bits(x):
            u = lax.bitcast_convert_type(x, jnp.int32)
            r = u + rbias + (lax.shift_right_logical(u, 16) & one)
            return r & himask

        for j in range(_CHUNK):
            def grp_body(i, _, j=j):
                off = pl.multiple_of(i * 16, 16)
                w0 = b0[j, pl.ds(off, 16)]
                w1 = b1[j, pl.ds(off, 16)]
                w2 = b2[j, pl.ds(off, 16)]
                s_lo = (_lo_f32(w0) + _lo_f32(w1) + _lo_f32(w2)) * third
                s_hi = (_hi_f32(w0) + _hi_f32(w1) + _hi_f32(w2)) * third
                packed = (lax.shift_right_logical(_rtne_bits(s_lo), 16)
                          | _rtne_bits(s_hi))
                b0[j, pl.ds(off, 16)] = packed
                return 0

            lax.fori_loop(0, (D // 2) // 16, grp_body, 0, unroll=4)
        pltpu.sync_copy(b0, out_hbm.at[pl.ds(base, _CHUNK)])
        return 0

    lax.fori_loop(0, nchunks, chunk_body, 0)


def _gather_avg(hf, i0, i1, i2):
    import functools as _ft
    mesh = plsc.VectorSubcoreMesh(core_axis_name="c", subcore_axis_name="s")
    kfn = _ft.partial(
        pl.kernel,
        mesh=mesh,
        out_type=jax.ShapeDtypeStruct((F, D // 2), jnp.int32),
        scratch_types=[
            pltpu.VMEM((F // 32,), jnp.int32),
            pltpu.VMEM((F // 32,), jnp.int32),
            pltpu.VMEM((F // 32,), jnp.int32),
            pltpu.VMEM((_CHUNK, D // 2), jnp.int32),
            pltpu.VMEM((_CHUNK, D // 2), jnp.int32),
            pltpu.VMEM((_CHUNK, D // 2), jnp.int32),
            pltpu.SemaphoreType.DMA,
        ],
    )(_gather_avg_body)
    return kfn(hf, i0, i1, i2)


def kernel(h, W1, b1, W2, b2, Wv, Wt1, bt1, Wt2, bt2, Wo):
    hf = h.reshape(F, D)
    p0, p1, p2, tab = _compute_pointers(hf, W1, b1, W2, b2)
    i0, i1, i2 = p0.reshape(F), p1.reshape(F), p2.reshape(F)
    gi = _gather_avg(tab, i0, i1, i2)
    M1, M2, M3, c0 = _prep_weights(Wv, Wt1, Wt2, Wo, bt2)
    z = _dense(tab, gi, M1, M2, M3, bt1, c0)
    return z.reshape(B, N, D)


# speedup vs baseline: 3.9426x; 1.0106x over previous
import functools

import jax
import jax.numpy as jnp
from jax import lax
from jax.experimental import pallas as pl
from jax.experimental.pallas import tpu as pltpu

def _gelu_exact(x):
    return 0.5 * x * (1.0 + lax.erf(x * jnp.float32(0.7071067811865476)))


B, N, D = 2, 4096, 1024
F = 8192
THRESH = 0.3
BM_PTR = 1024
BM_C = 512


def _ptr_body(h_ref, W1_ref, b1_ref, w2_ref, b2_ref, p0_ref, p1_ref, p2_ref, tab_ref):
    blocks_per_batch = N // BM_PTR
    row_off = (pl.program_id(0) // blocks_per_batch) * N
    h = h_ref[...]
    W1 = W1_ref[...].astype(jnp.bfloat16)
    b1 = b1_ref[...]
    w2 = w2_ref[...].astype(jnp.bfloat16)
    b2 = b2_ref[0, 0]
    outs = (p0_ref, p1_ref, p2_ref)
    u = lax.bitcast_convert_type(h, jnp.int32)
    r = u + 0x7FFF + (lax.shift_right_logical(u, 16) & 1)
    tab_ref[...] = (lax.shift_right_logical(r[:, :D // 2], 16)
                    | (r[:, D // 2:] & jnp.int32(-65536)))
    nb = None
    for b in range(3):
        hb = h if b == 0 else h + jnp.float32(0.1 * b)
        u = jnp.dot(hb.astype(jnp.bfloat16), W1,
                    preferred_element_type=jnp.float32) + b1
        gl = _gelu_exact(u)
        logit = jnp.dot(gl.astype(jnp.bfloat16), w2,
                        preferred_element_type=jnp.float32) + b2
        s = jax.nn.sigmoid(logit)
        tgt = jnp.clip(jnp.round(s * (N - 1)).astype(jnp.int32), 0, N - 1)
        if b == 0:
            nb = jnp.clip(jnp.round(s / THRESH).astype(jnp.int32), 1, 3)
            ptr = tgt
        else:
            ptr = jnp.where(b < nb, tgt, 0)
        outs[b][...] = ptr + row_off


def _compute_pointers(hf, W1, b1, W2, b2):
    grid = (F // BM_PTR,)
    out = pl.pallas_call(
        _ptr_body,
        grid=grid,
        in_specs=[
            pl.BlockSpec((BM_PTR, D), lambda i: (i, 0)),
            pl.BlockSpec((D, D // 2), lambda i: (0, 0)),
            pl.BlockSpec((1, D // 2), lambda i: (0, 0)),
            pl.BlockSpec((D // 2, 1), lambda i: (0, 0)),
            pl.BlockSpec((1, 1), lambda i: (0, 0)),
        ],
        out_specs=[pl.BlockSpec((BM_PTR, 1), lambda i: (i, 0))] * 3
        + [pl.BlockSpec((BM_PTR, D // 2), lambda i: (i, 0))],
        out_shape=[jax.ShapeDtypeStruct((F, 1), jnp.int32)] * 3
        + [jax.ShapeDtypeStruct((F, D // 2), jnp.int32)],
    )(hf, W1, b1.reshape(1, -1), W2, b2.reshape(1, 1))
    return out


def _prep_body(Wv_ref, Wt1_ref, Wt2_ref, Wo_ref, bt2_ref,
               M1_ref, M2_ref, M3_ref, c0_ref):
    Wv = Wv_ref[...].astype(jnp.bfloat16)
    Wo = Wo_ref[...].astype(jnp.bfloat16)
    M1_ref[...] = jnp.dot(Wv, Wt1_ref[:D, :].astype(jnp.bfloat16), preferred_element_type=jnp.float32)
    M2_ref[...] = jnp.dot(Wv, Wt1_ref[D:, :].astype(jnp.bfloat16), preferred_element_type=jnp.float32)
    M3_ref[...] = jnp.dot(Wt2_ref[...].astype(jnp.bfloat16), Wo, preferred_element_type=jnp.float32)
    c0_ref[...] = jnp.dot(bt2_ref[...].astype(jnp.bfloat16), Wo, preferred_element_type=jnp.float32)


def _prep_weights(Wv, Wt1, Wt2, Wo, bt2):
    return pl.pallas_call(
        _prep_body,
        out_shape=[
            jax.ShapeDtypeStruct((D, D), jnp.float32),
            jax.ShapeDtypeStruct((D, D), jnp.float32),
            jax.ShapeDtypeStruct((D, D), jnp.float32),
            jax.ShapeDtypeStruct((1, D), jnp.float32),
        ],
    )(Wv, Wt1, Wt2, Wo, bt2.reshape(1, -1))


def _unlo(w):
    return lax.bitcast_convert_type(lax.shift_left(w, 16), jnp.float32)


def _unhi(w):
    return lax.bitcast_convert_type(w & jnp.int32(-65536), jnp.float32)


def _dense_body(h_ref, g_ref, M1_ref, M2_ref, M3_ref, bt1_ref, c0_ref, o_ref):
    t = h_ref[...]
    gw = g_ref[...]
    Dh = D // 2
    pre = (jnp.dot(_unlo(t).astype(jnp.bfloat16),
                   M1_ref[:Dh, :].astype(jnp.bfloat16),
                   preferred_element_type=jnp.float32)
           + jnp.dot(_unhi(t).astype(jnp.bfloat16),
                     M1_ref[Dh:, :].astype(jnp.bfloat16),
                     preferred_element_type=jnp.float32)
           + jnp.dot(_unlo(gw).astype(jnp.bfloat16),
                     M2_ref[:Dh, :].astype(jnp.bfloat16),
                     preferred_element_type=jnp.float32)
           + jnp.dot(_unhi(gw).astype(jnp.bfloat16),
                     M2_ref[Dh:, :].astype(jnp.bfloat16),
                     preferred_element_type=jnp.float32)
           + bt1_ref[...])
    act = _gelu_exact(pre)
    o_ref[...] = jnp.dot(act.astype(jnp.bfloat16), M3_ref[...].astype(jnp.bfloat16),
                         preferred_element_type=jnp.float32) + c0_ref[...]


def _dense(hf, g, M1, M2, M3, bt1, c0):
    grid = (F // BM_C,)
    return pl.pallas_call(
        _dense_body,
        grid=grid,
        in_specs=[
            pl.BlockSpec((BM_C, D // 2), lambda i: (i, 0)),
            pl.BlockSpec((BM_C, D // 2), lambda i: (i, 0)),
            pl.BlockSpec((D, D), lambda i: (0, 0)),
            pl.BlockSpec((D, D), lambda i: (0, 0)),
            pl.BlockSpec((D, D), lambda i: (0, 0)),
            pl.BlockSpec((1, D), lambda i: (0, 0)),
            pl.BlockSpec((1, D), lambda i: (0, 0)),
        ],
        out_specs=pl.BlockSpec((BM_C, D), lambda i: (i, 0)),
        out_shape=jax.ShapeDtypeStruct((F, D), jnp.float32),
    )(hf, g, M1, M2, M3, bt1.reshape(1, -1), c0)


try:
    from jax.experimental.pallas import tpu_sc as plsc
    _SC_INFO = None

    def _sc_info():
        global _SC_INFO
        if _SC_INFO is None:
            _SC_INFO = plsc.get_sparse_core_info()
        return _SC_INFO
except ImportError:
    plsc = None

_CHUNK = 16


def _gather_avg_body(tab_hbm, i0_hbm, i1_hbm, i2_hbm, out_hbm,
                     idx0_v, idx1_v, idx2_v,
                     a0, a1, a2, b0, b1, b2, oa, ob,
                     gsa, gsb, wsa, wsb):
    nc = 2
    wid = lax.axis_index("s") * nc + lax.axis_index("c")
    nw = 32
    rows_per_w = F // nw
    nchunks = rows_per_w // _CHUNK
    row0 = wid * rows_per_w
    DW = D // 2

    pltpu.sync_copy(i0_hbm.at[pl.ds(row0, rows_per_w)], idx0_v)
    pltpu.sync_copy(i1_hbm.at[pl.ds(row0, rows_per_w)], idx1_v)
    pltpu.sync_copy(i2_hbm.at[pl.ds(row0, rows_per_w)], idx2_v)

    idxs = (idx0_v, idx1_v, idx2_v)
    bufs = ((a0, a1, a2), (b0, b1, b2))
    obufs = (oa, ob)
    gsems = (gsa, gsb)
    wsems = (wsa, wsb)

    third = jnp.full((16,), 1.0 / 3.0, dtype=jnp.float32)
    himask = jnp.full((16,), -65536, dtype=jnp.int32)
    rbias = jnp.full((16,), 0x7FFF, dtype=jnp.int32)
    one = jnp.full((16,), 1, dtype=jnp.int32)

    def _lo_f32(w):
        return lax.bitcast_convert_type(lax.shift_left(w, 16), jnp.float32)

    def _hi_f32(w):
        return lax.bitcast_convert_type(w & himask, jnp.float32)

    def _rtne_bits(x):
        u = lax.bitcast_convert_type(x, jnp.int32)
        r = u + rbias + (lax.shift_right_logical(u, 16) & one)
        return r & himask

    def _fire_gathers(k, s):
        koff = k * _CHUNK
        for br in range(3):
            pltpu.async_copy(tab_hbm.at[idxs[br].at[pl.ds(koff, _CHUNK)]],
                             bufs[s][br], gsems[s])

    def _wait_gathers(s):
        for br in range(3):
            pltpu.make_async_copy(tab_hbm.at[idxs[br].at[pl.ds(0, _CHUNK)]],
                                  bufs[s][br], gsems[s]).wait()

    def _wait_wb(s):
        pltpu.make_async_copy(obufs[s], out_hbm.at[pl.ds(row0, _CHUNK)],
                              wsems[s]).wait()

    _fire_gathers(0, 0)
    _fire_gathers(1, 1)

    def pair_body(p, _):
        for s in range(2):
            k = p * 2 + s
            _wait_gathers(s)

            @pl.when(k >= 2)
            def _():
                _wait_wb(s)

            bb0, bb1, bb2 = bufs[s]
            ov = obufs[s]

            def row_body(j, _):
                def grp_body(i, _):
                    off = pl.multiple_of(i * 16, 16)
                    w0 = bb0[j, pl.ds(off, 16)]
                    w1 = bb1[j, pl.ds(off, 16)]
                    w2 = bb2[j, pl.ds(off, 16)]
                    s_lo = (_lo_f32(w0) + _lo_f32(w1) + _lo_f32(w2)) * third
                    s_hi = (_hi_f32(w0) + _hi_f32(w1) + _hi_f32(w2)) * third
                    ov[j, pl.ds(off, 16)] = (
                        lax.shift_right_logical(_rtne_bits(s_lo), 16)
                        | _rtne_bits(s_hi))
                    return 0

                return lax.fori_loop(0, DW // 16, grp_body, 0, unroll=4)

            lax.fori_loop(0, _CHUNK, row_body, 0)
            base = pl.multiple_of(row0 + k * _CHUNK, _CHUNK)
            pltpu.async_copy(ov, out_hbm.at[pl.ds(base, _CHUNK)], wsems[s])

            @pl.when(k + 2 < nchunks)
            def _():
                _fire_gathers(k + 2, s)
        return 0

    lax.fori_loop(0, nchunks // 2, pair_body, 0)
    _wait_wb(0)
    _wait_wb(1)


def _gather_avg(hf, i0, i1, i2):
    import functools as _ft
    mesh = plsc.VectorSubcoreMesh(core_axis_name="c", subcore_axis_name="s")
    kfn = _ft.partial(
        pl.kernel,
        mesh=mesh,
        out_type=jax.ShapeDtypeStruct((F, D // 2), jnp.int32),
        scratch_types=[
            pltpu.VMEM((F // 32,), jnp.int32),
            pltpu.VMEM((F // 32,), jnp.int32),
            pltpu.VMEM((F // 32,), jnp.int32),
            pltpu.VMEM((_CHUNK, D // 2), jnp.int32),
            pltpu.VMEM((_CHUNK, D // 2), jnp.int32),
            pltpu.VMEM((_CHUNK, D // 2), jnp.int32),
            pltpu.VMEM((_CHUNK, D // 2), jnp.int32),
            pltpu.VMEM((_CHUNK, D // 2), jnp.int32),
            pltpu.VMEM((_CHUNK, D // 2), jnp.int32),
            pltpu.VMEM((_CHUNK, D // 2), jnp.int32),
            pltpu.VMEM((_CHUNK, D // 2), jnp.int32),
            pltpu.SemaphoreType.DMA,
            pltpu.SemaphoreType.DMA,
            pltpu.SemaphoreType.DMA,
            pltpu.SemaphoreType.DMA,
        ],
    )(_gather_avg_body)
    return kfn(hf, i0, i1, i2)


def kernel(h, W1, b1, W2, b2, Wv, Wt1, bt1, Wt2, bt2, Wo):
    hf = h.reshape(F, D)
    p0, p1, p2, tab = _compute_pointers(hf, W1, b1, W2, b2)
    i0, i1, i2 = p0.reshape(F), p1.reshape(F), p2.reshape(F)
    gi = _gather_avg(tab, i0, i1, i2)
    M1, M2, M3, c0 = _prep_weights(Wv, Wt1, Wt2, Wo, bt2)
    z = _dense(tab, gi, M1, M2, M3, bt1, c0)
    return z.reshape(B, N, D)
